# Initial kernel scaffold; baseline (speedup 1.0000x reference)
import jax
import jax.numpy as jnp
from jax.experimental import pallas as pl


def kernel(input_ids, table, norm_weight):
    raise NotImplementedError("write your pallas kernel here")


# speedup vs baseline: 6.1549x; 6.1549x over previous
import functools

import jax
import jax.numpy as jnp
from jax import lax
from jax.experimental import pallas as pl
from jax.experimental.pallas import tpu as pltpu
from jax.experimental.pallas import tpu_sc as plsc

_VOCAB = 100000
_DIM = 128
_EPS = 1e-12

_NORM_BLOCK = 2000

_NC = 2
_NS = 16
_NW = _NC * _NS
_G = 128


def _norm_body(t_ref, w_ref, o_ref):
    x = t_ref[...]
    ms = jnp.mean(x * x, axis=-1, keepdims=True)
    o_ref[...] = x * lax.rsqrt(ms + _EPS) * w_ref[...]


def _normalize_table(table, norm_weight):
    return pl.pallas_call(
        _norm_body,
        grid=(_VOCAB // _NORM_BLOCK,),
        in_specs=[
            pl.BlockSpec((_NORM_BLOCK, _DIM), lambda i: (i, 0)),
            pl.BlockSpec((1, _DIM), lambda i: (0, 0)),
        ],
        out_specs=pl.BlockSpec((_NORM_BLOCK, _DIM), lambda i: (i, 0)),
        out_shape=jax.ShapeDtypeStruct((_VOCAB, _DIM), jnp.float32),
    )(table, norm_weight.reshape(1, _DIM))


def _make_gather(n_ids):
    assert n_ids % (_NW * _G) == 0
    b_per_w = n_ids // _NW
    n_groups = b_per_w // _G
    mesh = plsc.VectorSubcoreMesh(
        core_axis_name="c", subcore_axis_name="s",
        num_cores=_NC, num_subcores=_NS,
    )

    @functools.partial(
        pl.kernel,
        out_type=jax.ShapeDtypeStruct((n_ids, _DIM), jnp.float32),
        mesh=mesh,
        scratch_types=[
            pltpu.VMEM((b_per_w,), jnp.int32),
            pltpu.VMEM((_G, _DIM), jnp.float32),
            pltpu.SemaphoreType.DMA,
        ],
    )
    def gather_kernel(tab_hbm, ids_hbm, out_hbm, idx_v, rows_v, sem):
        wid = lax.axis_index("s") * _NC + lax.axis_index("c")
        base = wid * b_per_w
        pltpu.sync_copy(ids_hbm.at[pl.ds(base, b_per_w)], idx_v)

        def body(g, carry):
            pltpu.async_copy(
                tab_hbm.at[idx_v.at[pl.ds(g * _G, _G)]], rows_v, sem
            ).wait()
            pltpu.sync_copy(rows_v, out_hbm.at[pl.ds(base + g * _G, _G)])
            return carry

        lax.fori_loop(0, n_groups, body, 0)

    return gather_kernel


def kernel(input_ids, table, norm_weight):
    b, l = input_ids.shape
    normed = _normalize_table(table, norm_weight)
    ids_flat = input_ids.reshape(-1)
    out = _make_gather(ids_flat.size)(normed, ids_flat)
    return out.reshape(b, l, _DIM)


# speedup vs baseline: 8.5681x; 1.3921x over previous
import functools

import jax
import jax.numpy as jnp
from jax import lax
from jax.experimental import pallas as pl
from jax.experimental.pallas import tpu as pltpu
from jax.experimental.pallas import tpu_sc as plsc

_VOCAB = 100000
_DIM = 128
_EPS = 1e-12

_NORM_BLOCK = 2000

_NC = 2
_NS = 16
_NW = _NC * _NS
_G = 128


def _norm_body(t_ref, w_ref, o_ref):
    x = t_ref[...]
    ms = jnp.mean(x * x, axis=-1, keepdims=True)
    o_ref[...] = x * lax.rsqrt(ms + _EPS) * w_ref[...]


def _normalize_table(table, norm_weight):
    return pl.pallas_call(
        _norm_body,
        grid=(_VOCAB // _NORM_BLOCK,),
        in_specs=[
            pl.BlockSpec((_NORM_BLOCK, _DIM), lambda i: (i, 0)),
            pl.BlockSpec((1, _DIM), lambda i: (0, 0)),
        ],
        out_specs=pl.BlockSpec((_NORM_BLOCK, _DIM), lambda i: (i, 0)),
        out_shape=jax.ShapeDtypeStruct((_VOCAB, _DIM), jnp.float32),
    )(table, norm_weight.reshape(1, _DIM))


_NBUF = 4


def _make_gather(n_ids):
    assert n_ids % (_NW * _G * _NBUF) == 0
    b_per_w = n_ids // _NW
    n_groups = b_per_w // _G
    mesh = plsc.VectorSubcoreMesh(
        core_axis_name="c", subcore_axis_name="s",
        num_cores=_NC, num_subcores=_NS,
    )

    @functools.partial(
        pl.kernel,
        out_type=jax.ShapeDtypeStruct((n_ids, _DIM), jnp.float32),
        mesh=mesh,
        scratch_types=[
            pltpu.VMEM((b_per_w,), jnp.int32),
            pltpu.VMEM((_NBUF, _G, _DIM), jnp.float32),
            pltpu.SemaphoreType.DMA((_NBUF,)),
            pltpu.SemaphoreType.DMA((_NBUF,)),
        ],
    )
    def gather_kernel(tab_hbm, ids_hbm, out_hbm, idx_v, rows_v, gsem, wsem):
        wid = lax.axis_index("s") * _NC + lax.axis_index("c")
        base = wid * b_per_w
        pltpu.sync_copy(ids_hbm.at[pl.ds(base, b_per_w)], idx_v)

        def start_gather(b, g):
            pltpu.async_copy(
                tab_hbm.at[idx_v.at[pl.ds(g * _G, _G)]],
                rows_v.at[b], gsem.at[b],
            )

        def wait_gather(b, g):
            pltpu.make_async_copy(
                tab_hbm.at[idx_v.at[pl.ds(g * _G, _G)]],
                rows_v.at[b], gsem.at[b],
            ).wait()

        def start_write(b, g):
            pltpu.async_copy(
                rows_v.at[b], out_hbm.at[pl.ds(base + g * _G, _G)], wsem.at[b]
            )

        def wait_write(b, g):
            pltpu.make_async_copy(
                rows_v.at[b], out_hbm.at[pl.ds(base + g * _G, _G)], wsem.at[b]
            ).wait()

        for b in range(_NBUF):
            start_gather(b, b)

        def outer(it, carry):
            g0 = it * _NBUF
            for b in range(_NBUF):
                g = g0 + b
                wait_gather(b, g)
                start_write(b, g)
                wait_write(b, g)
                start_gather(b, g + _NBUF)
            return carry

        lax.fori_loop(0, n_groups // _NBUF - 1, outer, 0)

        for b in range(_NBUF):
            g = n_groups - _NBUF + b
            wait_gather(b, g)
            start_write(b, g)
        for b in range(_NBUF):
            g = n_groups - _NBUF + b
            wait_write(b, g)

    return gather_kernel


def kernel(input_ids, table, norm_weight):
    b, l = input_ids.shape
    normed = _normalize_table(table, norm_weight)
    ids_flat = input_ids.reshape(-1)
    out = _make_gather(ids_flat.size)(normed, ids_flat)
    return out.reshape(b, l, _DIM)
